# baseline (speedup 1.0000x reference)
import functools

import jax
import jax.numpy as jnp
from jax import lax
from jax.experimental import pallas as pl
from jax.experimental.pallas import tpu as pltpu
from jax.experimental.pallas import tpu_sc as plsc

B, C, N, K = 8, 128, 1024, 16
BLK = 256


def _topk_body(cloud_ref, idx_ref):
    i = pl.program_id(1)
    xf = cloud_ref[0]
    rows = cloud_ref[0, :, pl.ds(i * BLK, BLK)]
    inner = lax.dot_general(
        rows, xf, (((0,), (0,)), ((), ())),
        preferred_element_type=jnp.float32)
    sq = jnp.sum(xf * xf, axis=0)
    sq_rows = jnp.sum(rows * rows, axis=0)
    d = sq_rows[:, None] + sq[None, :] - 2.0 * inner

    lane = lax.broadcasted_iota(jnp.int32, (BLK, N), 1)
    cols = []
    for _ in range(K):
        m = jnp.min(d, axis=1)
        cand = jnp.where(d == m[:, None], lane, N)
        amin = jnp.min(cand, axis=1)
        cols.append(amin)
        d = jnp.where(lane == amin[:, None], jnp.inf, d)
    idx_ref[0] = jnp.stack(cols, axis=1)


def _nn_idx(cloud):
    return pl.pallas_call(
        _topk_body,
        grid=(B, N // BLK),
        in_specs=[pl.BlockSpec((1, C, N), lambda b, i: (b, 0, 0))],
        out_specs=pl.BlockSpec((1, BLK, K), lambda b, i: (b, i, 0)),
        out_shape=jax.ShapeDtypeStruct((B, N, K), jnp.int32),
    )(cloud)


_NW = 32
_CPW = C * B // _NW


@functools.lru_cache(maxsize=None)
def _edge_sc():
    mesh = plsc.VectorSubcoreMesh(
        core_axis_name="c", subcore_axis_name="s", num_cores=2,
        num_subcores=16)

    @functools.partial(
        pl.kernel,
        out_type=jax.ShapeDtypeStruct((B, 2 * C, N * K), jnp.float32),
        mesh=mesh,
        compiler_params=pltpu.CompilerParams(needs_layout_passes=False),
        scratch_types=[
            pltpu.VMEM((N * K,), jnp.int32),
            pltpu.VMEM((N,), jnp.float32),
            pltpu.VMEM((N,), jnp.float32),
            pltpu.VMEM((N * K,), jnp.float32),
            pltpu.VMEM((N * K,), jnp.float32),
            pltpu.VMEM((N * K,), jnp.float32),
            pltpu.VMEM((N * K,), jnp.float32),
            pltpu.SemaphoreType.DMA,
            pltpu.SemaphoreType.DMA,
        ],
    )
    def edge_sc(cloud_hbm, idx_hbm, out_hbm, idx_v,
                row0, row1, cen0, cen1, edge0, edge1, sem0, sem1):
        rows, cens, edges, sems = (row0, row1), (cen0, cen1), (edge0, edge1), \
            (sem0, sem1)
        wid = lax.axis_index("s") * 2 + lax.axis_index("c")
        b = wid // (_NW // B)
        c0 = (wid % (_NW // B)) * _CPW
        pltpu.sync_copy(idx_hbm.at[b], idx_v)

        iota16 = lax.iota(jnp.int32, K)
        pending = [None, None]

        for cc in range(_CPW):
            s = cc % 2
            c = c0 + cc
            if pending[s] is not None:
                for d in pending[s]:
                    d.wait()
            pltpu.sync_copy(cloud_hbm.at[b, c], rows[s])
            row_s, cen_s, edge_s = rows[s], cens[s], edges[s]

            def chunk(t, _, row_s=row_s, cen_s=cen_s, edge_s=edge_s):
                base = t * K
                cvec = row_s[pl.ds(base, K)]
                rvec = base + iota16
                pos16 = rvec * K
                for j in range(K):
                    posj = pos16 + j
                    iv = plsc.load_gather(idx_v, [posj])
                    nb = plsc.load_gather(row_s, [iv])
                    plsc.store_scatter(cen_s, [posj], cvec)
                    plsc.store_scatter(edge_s, [posj], nb - cvec)
                return 0

            lax.fori_loop(0, N // K, chunk, 0)
            d1 = pltpu.async_copy(cen_s, out_hbm.at[b, c], sems[s])
            d2 = pltpu.async_copy(edge_s, out_hbm.at[b, C + c], sems[s])
            pending[s] = (d1, d2)

        for s in (0, 1):
            for d in pending[s]:
                d.wait()

    return edge_sc


def kernel(cloud):
    idx = _nn_idx(cloud)
    out = _edge_sc()(cloud, idx.reshape(B, N * K))
    return out.reshape(B, 2 * C, N, K)


# speedup vs baseline: 2.0971x; 2.0971x over previous
import functools

import jax
import jax.numpy as jnp
from jax import lax
from jax.experimental import pallas as pl
from jax.experimental.pallas import tpu as pltpu
from jax.experimental.pallas import tpu_sc as plsc

B, C, N, K = 8, 128, 1024, 16
BLK = 256


def _topk_body(cloud_ref, idx_ref):
    i = pl.program_id(1)
    xf = cloud_ref[0]
    rows = cloud_ref[0, :, pl.ds(i * BLK, BLK)]
    inner = lax.dot_general(
        rows, xf, (((0,), (0,)), ((), ())),
        preferred_element_type=jnp.float32)
    sq = jnp.sum(xf * xf, axis=0)
    sq_rows = jnp.sum(rows * rows, axis=0)
    d = sq_rows[:, None] + sq[None, :] - 2.0 * inner

    lane = lax.broadcasted_iota(jnp.int32, (BLK, N), 1)
    cols = []
    for _ in range(K):
        m = jnp.min(d, axis=1)
        cand = jnp.where(d == m[:, None], lane, N)
        amin = jnp.min(cand, axis=1)
        cols.append(amin)
        d = jnp.where(lane == amin[:, None], jnp.inf, d)
    idx_ref[0] = jnp.stack(cols, axis=1)


def _nn_idx(cloud):
    return pl.pallas_call(
        _topk_body,
        grid=(B, N // BLK),
        in_specs=[pl.BlockSpec((1, C, N), lambda b, i: (b, 0, 0))],
        out_specs=pl.BlockSpec((1, BLK, K), lambda b, i: (b, i, 0)),
        out_shape=jax.ShapeDtypeStruct((B, N, K), jnp.int32),
    )(cloud)


_NW = 32
_CPW = C * B // _NW


@functools.lru_cache(maxsize=None)
def _edge_sc():
    mesh = plsc.VectorSubcoreMesh(
        core_axis_name="c", subcore_axis_name="s", num_cores=2,
        num_subcores=16)

    @functools.partial(
        pl.kernel,
        out_type=jax.ShapeDtypeStruct((B, 2 * C, N * K), jnp.float32),
        mesh=mesh,
        compiler_params=pltpu.CompilerParams(needs_layout_passes=False),
        scratch_types=[
            pltpu.VMEM((N * K,), jnp.int32),
            pltpu.VMEM((N,), jnp.float32),
            pltpu.VMEM((N,), jnp.float32),
            pltpu.VMEM((N * K,), jnp.float32),
            pltpu.VMEM((N * K,), jnp.float32),
            pltpu.VMEM((N * K,), jnp.float32),
            pltpu.VMEM((N * K,), jnp.float32),
            pltpu.SemaphoreType.DMA,
            pltpu.SemaphoreType.DMA,
        ],
    )
    def edge_sc(cloud_hbm, idx_hbm, out_hbm, idx_v,
                row0, row1, cen0, cen1, edge0, edge1, sem0, sem1):
        rows, cens, edges, sems = (row0, row1), (cen0, cen1), (edge0, edge1), \
            (sem0, sem1)
        wid = lax.axis_index("s") * 2 + lax.axis_index("c")
        b = wid // (_NW // B)
        c0 = (wid % (_NW // B)) * _CPW
        pltpu.sync_copy(idx_hbm.at[b], idx_v)

        zeros16 = jnp.zeros((K,), jnp.int32)
        pending = [None, None]

        for cc in range(_CPW):
            s = cc % 2
            c = c0 + cc
            if pending[s] is not None:
                for d in pending[s]:
                    d.wait()
            pltpu.sync_copy(cloud_hbm.at[b, c], rows[s])
            row_s, cen_s, edge_s = rows[s], cens[s], edges[s]

            @plsc.parallel_loop(0, N, 1, unroll=8)
            def per_vec(i, row_s=row_s, cen_s=cen_s, edge_s=edge_s):
                iv = idx_v[pl.ds(i * K, K)]
                nb = plsc.load_gather(row_s, [iv])
                cv = plsc.load_gather(row_s, [zeros16 + i])
                cen_s[pl.ds(i * K, K)] = cv
                edge_s[pl.ds(i * K, K)] = nb - cv
            d1 = pltpu.async_copy(cen_s, out_hbm.at[b, c], sems[s])
            d2 = pltpu.async_copy(edge_s, out_hbm.at[b, C + c], sems[s])
            pending[s] = (d1, d2)

        for s in (0, 1):
            for d in pending[s]:
                d.wait()

    return edge_sc


def kernel(cloud):
    idx = _nn_idx(cloud)
    out = _edge_sc()(cloud, idx.reshape(B, N * K))
    return out.reshape(B, 2 * C, N, K)


# speedup vs baseline: 3.3251x; 1.5856x over previous
import functools

import jax
import jax.numpy as jnp
from jax import lax
from jax.experimental import pallas as pl
from jax.experimental.pallas import tpu as pltpu
from jax.experimental.pallas import tpu_sc as plsc

B, C, N, K = 8, 128, 1024, 16
BLK = 256


def _topk_body(cloud_ref, idx_ref):
    i = pl.program_id(1)
    xf = cloud_ref[0]
    rows = cloud_ref[0, :, pl.ds(i * BLK, BLK)]
    inner = lax.dot_general(
        rows, xf, (((0,), (0,)), ((), ())),
        preferred_element_type=jnp.float32)
    sq = jnp.sum(xf * xf, axis=0)
    sq_rows = jnp.sum(rows * rows, axis=0)
    d = sq_rows[:, None] + sq[None, :] - 2.0 * inner

    lane = lax.broadcasted_iota(jnp.int32, (BLK, N), 1)
    cols = []
    for _ in range(K):
        m = jnp.min(d, axis=1)
        cand = jnp.where(d == m[:, None], lane, N)
        amin = jnp.min(cand, axis=1)
        cols.append(amin)
        d = jnp.where(lane == amin[:, None], jnp.inf, d)
    idx_ref[0] = jnp.stack(cols, axis=1)


def _nn_idx(cloud):
    return pl.pallas_call(
        _topk_body,
        grid=(B, N // BLK),
        in_specs=[pl.BlockSpec((1, C, N), lambda b, i: (b, 0, 0))],
        out_specs=pl.BlockSpec((1, BLK, K), lambda b, i: (b, i, 0)),
        out_shape=jax.ShapeDtypeStruct((B, N, K), jnp.int32),
    )(cloud)


_NW = 32
_CPW = C * B // _NW


@functools.lru_cache(maxsize=None)
def _edge_sc():
    mesh = plsc.VectorSubcoreMesh(
        core_axis_name="c", subcore_axis_name="s", num_cores=2,
        num_subcores=16)

    @functools.partial(
        pl.kernel,
        out_type=jax.ShapeDtypeStruct((B, 2 * C, K, N), jnp.float32),
        mesh=mesh,
        compiler_params=pltpu.CompilerParams(needs_layout_passes=False),
        scratch_types=[
            pltpu.VMEM((N * K,), jnp.int32),
            pltpu.VMEM((N,), jnp.float32),
            pltpu.VMEM((N,), jnp.float32),
            pltpu.VMEM((K, N), jnp.float32),
            pltpu.VMEM((K, N), jnp.float32),
            pltpu.VMEM((K, N), jnp.float32),
            pltpu.VMEM((K, N), jnp.float32),
            pltpu.SemaphoreType.DMA,
            pltpu.SemaphoreType.DMA,
        ],
    )
    def edge_sc(cloud_hbm, idx_hbm, out_hbm, idx_v,
                row0, row1, cen0, cen1, edge0, edge1, sem0, sem1):
        rows, cens, edges, sems = (row0, row1), (cen0, cen1), (edge0, edge1), \
            (sem0, sem1)
        wid = lax.axis_index("s") * 2 + lax.axis_index("c")
        b = wid // (_NW // B)
        c0 = (wid % (_NW // B)) * _CPW
        pltpu.sync_copy(idx_hbm.at[b], idx_v)

        iotax16 = lax.iota(jnp.int32, K) * K
        pending = [None, None]

        for cc in range(_CPW):
            s = cc % 2
            c = c0 + cc
            if pending[s] is not None:
                for d in pending[s]:
                    d.wait()
            pltpu.sync_copy(cloud_hbm.at[b, c], rows[s])
            row_s, cen_s, edge_s = rows[s], cens[s], edges[s]

            @plsc.parallel_loop(0, N, 1, unroll=8)
            def per_vec(v, row_s=row_s, cen_s=cen_s, edge_s=edge_s):
                i = v // K
                k = v % K
                nsl = pl.ds(i * K, K)
                cvec = row_s[nsl]
                iv = plsc.load_gather(idx_v, [i * (K * K) + iotax16 + k])
                nb = plsc.load_gather(row_s, [iv])
                cen_s[k, nsl] = cvec
                edge_s[k, nsl] = nb - cvec
            d1 = pltpu.async_copy(cen_s, out_hbm.at[b, c], sems[s])
            d2 = pltpu.async_copy(edge_s, out_hbm.at[b, C + c], sems[s])
            pending[s] = (d1, d2)

        for s in (0, 1):
            for d in pending[s]:
                d.wait()

    return edge_sc


def kernel(cloud):
    idx = _nn_idx(cloud)
    out = _edge_sc()(cloud, idx.reshape(B, N * K))
    return jnp.transpose(out, (0, 1, 3, 2))


# speedup vs baseline: 4.1371x; 1.2442x over previous
import functools

import jax
import jax.numpy as jnp
from jax import lax
from jax.experimental import pallas as pl
from jax.experimental.pallas import tpu as pltpu
from jax.experimental.pallas import tpu_sc as plsc

B, C, N, K = 8, 128, 1024, 16
BLK = 256


def _topk_body(cloud_ref, idx_ref):
    i = pl.program_id(1)
    xf = cloud_ref[0]
    rows = cloud_ref[0, :, pl.ds(i * BLK, BLK)]
    inner = lax.dot_general(
        rows, xf, (((0,), (0,)), ((), ())),
        preferred_element_type=jnp.float32)
    sq = jnp.sum(xf * xf, axis=0)
    sq_rows = jnp.sum(rows * rows, axis=0)
    d = sq_rows[:, None] + sq[None, :] - 2.0 * inner

    lanef = lax.broadcasted_iota(jnp.int32, (BLK, N), 1).astype(jnp.float32)
    cols = []
    for _ in range(K):
        m = jnp.min(d, axis=1)
        candf = jnp.where(d == m[:, None], lanef, float(N))
        aminf = jnp.min(candf, axis=1)
        cols.append(aminf.astype(jnp.int32))
        d = jnp.where(lanef == aminf[:, None], jnp.inf, d)
    idx_ref[0] = jnp.stack(cols, axis=1)


def _nn_idx(cloud):
    return pl.pallas_call(
        _topk_body,
        grid=(B, N // BLK),
        in_specs=[pl.BlockSpec((1, C, N), lambda b, i: (b, 0, 0))],
        out_specs=pl.BlockSpec((1, BLK, K), lambda b, i: (b, i, 0)),
        out_shape=jax.ShapeDtypeStruct((B, N, K), jnp.int32),
    )(cloud)


_NW = 32
_CPW = C * B // _NW


@functools.lru_cache(maxsize=None)
def _edge_sc():
    mesh = plsc.VectorSubcoreMesh(
        core_axis_name="c", subcore_axis_name="s", num_cores=2,
        num_subcores=16)

    @functools.partial(
        pl.kernel,
        out_type=jax.ShapeDtypeStruct((B, 2 * C, K, N), jnp.float32),
        mesh=mesh,
        compiler_params=pltpu.CompilerParams(needs_layout_passes=False),
        scratch_types=[
            pltpu.VMEM((N * K,), jnp.int32),
            pltpu.VMEM((N,), jnp.float32),
            pltpu.VMEM((N,), jnp.float32),
            pltpu.VMEM((K, N), jnp.float32),
            pltpu.VMEM((K, N), jnp.float32),
            pltpu.VMEM((K, N), jnp.float32),
            pltpu.VMEM((K, N), jnp.float32),
            pltpu.SemaphoreType.DMA,
            pltpu.SemaphoreType.DMA,
        ],
    )
    def edge_sc(cloud_hbm, idx_hbm, out_hbm, idx_v,
                row0, row1, cen0, cen1, edge0, edge1, sem0, sem1):
        rows, cens, edges, sems = (row0, row1), (cen0, cen1), (edge0, edge1), \
            (sem0, sem1)
        wid = lax.axis_index("s") * 2 + lax.axis_index("c")
        b = wid // (_NW // B)
        c0 = (wid % (_NW // B)) * _CPW
        pltpu.sync_copy(idx_hbm.at[b], idx_v)

        iotax16 = lax.iota(jnp.int32, K) * K
        pending = [None, None]

        for cc in range(_CPW):
            s = cc % 2
            c = c0 + cc
            if pending[s] is not None:
                for d in pending[s]:
                    d.wait()
            pltpu.sync_copy(cloud_hbm.at[b, c], rows[s])
            row_s, cen_s, edge_s = rows[s], cens[s], edges[s]

            @plsc.parallel_loop(0, N, 1, unroll=8)
            def per_vec(v, row_s=row_s, cen_s=cen_s, edge_s=edge_s):
                i = v // K
                k = v % K
                nsl = pl.ds(i * K, K)
                cvec = row_s[nsl]
                iv = plsc.load_gather(idx_v, [i * (K * K) + iotax16 + k])
                nb = plsc.load_gather(row_s, [iv])
                cen_s[k, nsl] = cvec
                edge_s[k, nsl] = nb - cvec
            d1 = pltpu.async_copy(cen_s, out_hbm.at[b, c], sems[s])
            d2 = pltpu.async_copy(edge_s, out_hbm.at[b, C + c], sems[s])
            pending[s] = (d1, d2)

        for s in (0, 1):
            for d in pending[s]:
                d.wait()

    return edge_sc


def kernel(cloud):
    idx = _nn_idx(cloud)
    out = _edge_sc()(cloud, idx.reshape(B, N * K))
    return jnp.transpose(out, (0, 1, 3, 2))


# speedup vs baseline: 4.6398x; 1.1215x over previous
import functools

import jax
import jax.numpy as jnp
from jax import lax
from jax.experimental import pallas as pl
from jax.experimental.pallas import tpu as pltpu
from jax.experimental.pallas import tpu_sc as plsc

B, C, N, K = 8, 128, 1024, 16
BLK = 512


def _topk_body(cloud_ref, idx_ref):
    i = pl.program_id(1)
    xf = cloud_ref[0]
    rows = cloud_ref[0, :, pl.ds(i * BLK, BLK)]
    inner = lax.dot_general(
        rows, xf, (((0,), (0,)), ((), ())),
        preferred_element_type=jnp.float32)
    sq = jnp.sum(xf * xf, axis=0)
    sq_rows = jnp.sum(rows * rows, axis=0)
    d = sq_rows[:, None] + sq[None, :] - 2.0 * inner

    lanef = lax.broadcasted_iota(jnp.int32, (BLK, N), 1).astype(jnp.float32)
    cols = []
    for _ in range(K):
        m = jnp.min(d, axis=1)
        candf = jnp.where(d == m[:, None], lanef, float(N))
        aminf = jnp.min(candf, axis=1)
        cols.append(aminf.astype(jnp.int32))
        d = jnp.where(lanef == aminf[:, None], jnp.inf, d)
    idx_ref[0] = jnp.stack(cols, axis=1)


def _nn_idx(cloud):
    return pl.pallas_call(
        _topk_body,
        grid=(B, N // BLK),
        in_specs=[pl.BlockSpec((1, C, N), lambda b, i: (b, 0, 0))],
        out_specs=pl.BlockSpec((1, BLK, K), lambda b, i: (b, i, 0)),
        out_shape=jax.ShapeDtypeStruct((B, N, K), jnp.int32),
    )(cloud)


_NW = 32
_CPW = C * B // _NW


@functools.lru_cache(maxsize=None)
def _edge_sc():
    mesh = plsc.VectorSubcoreMesh(
        core_axis_name="c", subcore_axis_name="s", num_cores=2,
        num_subcores=16)

    @functools.partial(
        pl.kernel,
        out_type=jax.ShapeDtypeStruct((B, 2 * C, K, N), jnp.float32),
        mesh=mesh,
        compiler_params=pltpu.CompilerParams(needs_layout_passes=False),
        scratch_types=[
            pltpu.VMEM((N * K,), jnp.int32),
            pltpu.VMEM((N,), jnp.float32),
            pltpu.VMEM((N,), jnp.float32),
            pltpu.VMEM((K, N), jnp.float32),
            pltpu.VMEM((K, N), jnp.float32),
            pltpu.VMEM((K, N), jnp.float32),
            pltpu.VMEM((K, N), jnp.float32),
            pltpu.SemaphoreType.DMA,
            pltpu.SemaphoreType.DMA,
            pltpu.SemaphoreType.DMA,
            pltpu.SemaphoreType.DMA,
        ],
    )
    def edge_sc(cloud_hbm, idx_hbm, out_hbm, idx_v,
                row0, row1, cen0, cen1, edge0, edge1,
                sem0, sem1, rsem0, rsem1):
        rows, cens, edges, sems = (row0, row1), (cen0, cen1), (edge0, edge1), \
            (sem0, sem1)
        rsems = (rsem0, rsem1)
        wid = lax.axis_index("s") * 2 + lax.axis_index("c")
        b = wid // (_NW // B)
        c0 = (wid % (_NW // B)) * _CPW
        pltpu.sync_copy(idx_hbm.at[b], idx_v)

        iotax16 = lax.iota(jnp.int32, K) * K
        pending = [None, None]
        rowdma = [None, None]
        rowdma[0] = pltpu.async_copy(cloud_hbm.at[b, c0], rows[0], rsems[0])

        for cc in range(_CPW):
            s = cc % 2
            c = c0 + cc
            rowdma[s].wait()
            if cc + 1 < _CPW:
                rowdma[1 - s] = pltpu.async_copy(
                    cloud_hbm.at[b, c + 1], rows[1 - s], rsems[1 - s])
            if pending[s] is not None:
                for d in pending[s]:
                    d.wait()
            row_s, cen_s, edge_s = rows[s], cens[s], edges[s]

            @plsc.parallel_loop(0, N, 1, unroll=8)
            def per_vec(v, row_s=row_s, cen_s=cen_s, edge_s=edge_s):
                i = v // K
                k = v % K
                nsl = pl.ds(i * K, K)
                cvec = row_s[nsl]
                iv = plsc.load_gather(idx_v, [i * (K * K) + iotax16 + k])
                nb = plsc.load_gather(row_s, [iv])
                cen_s[k, nsl] = cvec
                edge_s[k, nsl] = nb - cvec
            d1 = pltpu.async_copy(cen_s, out_hbm.at[b, c], sems[s])
            d2 = pltpu.async_copy(edge_s, out_hbm.at[b, C + c], sems[s])
            pending[s] = (d1, d2)

        for s in (0, 1):
            for d in pending[s]:
                d.wait()

    return edge_sc


def kernel(cloud):
    idx = _nn_idx(cloud)
    out = _edge_sc()(cloud, idx.reshape(B, N * K))
    return jnp.transpose(out, (0, 1, 3, 2))
